# baseline (speedup 1.0000x reference)
import functools

import jax
import jax.numpy as jnp
from jax import lax
from jax.experimental import pallas as pl
from jax.experimental.pallas import tpu as pltpu
from jax.experimental.pallas import tpu_sc as plsc

NC = 2
NS = 16
NW = NC * NS
CHUNK = 128
LANES = 16


def _sc_gather(table, idx, n_chunks, D):
    mesh = plsc.VectorSubcoreMesh(core_axis_name="c", subcore_axis_name="s")
    N = NW * n_chunks * CHUNK
    per_w = n_chunks * CHUNK

    @functools.partial(
        pl.kernel,
        mesh=mesh,
        compiler_params=pltpu.CompilerParams(use_tc_tiling_on_sc=False),
        out_type=jax.ShapeDtypeStruct((N, D), jnp.float32),
        scratch_types=[
            pltpu.VMEM((n_chunks, CHUNK), jnp.int32),
            pltpu.VMEM((2, CHUNK, D), jnp.float32),
            pltpu.SemaphoreType.DMA,
            pltpu.SemaphoreType.DMA,
        ],
    )
    def body(table_hbm, idx_hbm, out_hbm, idx_v, rows_v, gsem, ssem):
        wid = lax.axis_index("s") * NC + lax.axis_index("c")
        base = wid * per_w
        pltpu.sync_copy(idx_hbm.at[wid], idx_v)

        def gathers(g, buf):
            for t in range(CHUNK // LANES):
                v = idx_v[g, pl.ds(t * LANES, LANES)]
                for l in range(LANES):
                    pltpu.async_copy(
                        table_hbm.at[v[l]], rows_v.at[buf, t * LANES + l],
                        gsem)

        def gathers_wait(buf):
            pltpu.make_async_copy(
                table_hbm.at[pl.ds(0, CHUNK)], rows_v.at[buf], gsem).wait()

        def store_start(g):
            pltpu.async_copy(
                rows_v.at[lax.rem(g, 2)],
                out_hbm.at[pl.ds(base + g * CHUNK, CHUNK)], ssem)

        def store_wait(g):
            pltpu.make_async_copy(
                rows_v.at[lax.rem(g, 2)],
                out_hbm.at[pl.ds(base + g * CHUNK, CHUNK)], ssem).wait()

        gathers(0, 0)

        def step(g, carry):
            gathers_wait(lax.rem(g, 2))
            store_start(g)

            @pl.when(g >= 1)
            def _():
                store_wait(g - 1)

            @pl.when(g <= n_chunks - 2)
            def _():
                gathers(g + 1, lax.rem(g + 1, 2))

            return carry

        lax.fori_loop(0, n_chunks, step, 0)
        store_wait(n_chunks - 1)

    return body(table, idx)


def kernel(table, indices):
    V, D = table.shape
    B, H = indices.shape
    N = B * H
    assert N % (NW * CHUNK) == 0
    n_chunks = N // (NW * CHUNK)
    idx = indices.astype(jnp.int32).reshape(NW, n_chunks, CHUNK)
    out = _sc_gather(table, idx, n_chunks, D)
    return out.reshape(B, H, D)
